# baseline (speedup 1.0000x reference)
import functools

import jax
import jax.numpy as jnp
from jax import lax
from jax.experimental import pallas as pl
from jax.experimental.pallas import tpu as pltpu
from jax.experimental.pallas import tpu_sc as plsc

N = 10000
E = 320000
D_IN = 128
D_OUT = 64
D_HID = 128

NC = 2
NS = 16
NW = NC * NS
EPT = E // NW
CH = 80
NCH = EPT // CH
NB = 4
RPT = 624
RTAIL = N - NS * RPT


def _rowwise_copy(s, make_copy):
    make_copy(s * RPT, RPT)

    @pl.when(s == NS - 1)
    def _():
        make_copy(NS * RPT, RTAIL)


_MESH = plsc.VectorSubcoreMesh(core_axis_name="c", subcore_axis_name="s")


@functools.partial(
    pl.kernel,
    out_type=[jax.ShapeDtypeStruct((N,), jnp.float32),
              jax.ShapeDtypeStruct((N,), jnp.float32)],
    mesh=_MESH,
    scratch_types=[
        pltpu.VMEM((CH,), jnp.float32),
        pltpu.VMEM((2, CH), jnp.int32),
        pltpu.VMEM((640,), jnp.float32),
        pltpu.VMEM_SHARED((N,), jnp.float32),
        pltpu.SemaphoreType.DMA,
        pltpu.SemaphoreType.DMA,
    ],
)
def _deg_kernel(dst_hbm, out0, out1, ones_v, didx_v, slab_v, acc_sh,
                isem0, isem1):
    c = lax.axis_index("c")
    s = lax.axis_index("s")
    base = (c * NS + s) * EPT
    isem = (isem0, isem1)
    outs = (out0, out1)
    for k in range(CH // 16):
        ones_v[pl.ds(k * 16, 16)] = jnp.ones((16,), jnp.float32)
    for k in range(640 // 16):
        slab_v[pl.ds(k * 16, 16)] = jnp.zeros((16,), jnp.float32)
    pltpu.async_copy(dst_hbm.at[pl.ds(base, CH)], didx_v.at[0], isem[0])
    pltpu.sync_copy(slab_v.at[pl.ds(0, RPT)], acc_sh.at[pl.ds(s * RPT, RPT)])

    @pl.when(s == NS - 1)
    def _():
        pltpu.sync_copy(slab_v.at[pl.ds(0, RTAIL)], acc_sh.at[pl.ds(NS * RPT, RTAIL)])

    plsc.subcore_barrier()

    def chunk(j, carry):
        for b in (0, 1):

            @pl.when(lax.rem(j, 2) == b)
            def _():
                bn = 1 - b

                @pl.when(j + 1 < NCH)
                def _():
                    pltpu.async_copy(
                        dst_hbm.at[pl.ds(base + (j + 1) * CH, CH)],
                        didx_v.at[bn], isem[bn])

                pltpu.make_async_copy(
                    dst_hbm.at[pl.ds(base + j * CH, CH)],
                    didx_v.at[b], isem[b]).wait()
                pltpu.sync_copy(ones_v, acc_sh.at[didx_v.at[b]], add=True)

        return carry

    lax.fori_loop(0, NCH, chunk, 0)
    plsc.subcore_barrier()
    for cc in range(NC):

        @pl.when(c == cc)
        def _():
            pltpu.sync_copy(acc_sh.at[pl.ds(s * RPT, RPT)],
                            slab_v.at[pl.ds(0, RPT)])
            pltpu.sync_copy(slab_v.at[pl.ds(0, RPT)],
                            outs[cc].at[pl.ds(s * RPT, RPT)])

            @pl.when(s == NS - 1)
            def _():
                pltpu.sync_copy(acc_sh.at[pl.ds(NS * RPT, RTAIL)],
                                slab_v.at[pl.ds(0, RTAIL)])
                pltpu.sync_copy(slab_v.at[pl.ds(0, RTAIL)],
                                outs[cc].at[pl.ds(NS * RPT, RTAIL)])


@functools.partial(
    pl.kernel,
    out_type=jax.ShapeDtypeStruct((NC, N, D_HID), jnp.float32),
    mesh=_MESH,
    scratch_types=[
        pltpu.VMEM((NB, CH), jnp.int32),
        pltpu.VMEM((NB, CH), jnp.int32),
        pltpu.VMEM((NB, CH, D_HID), jnp.float32),
        pltpu.VMEM_SHARED((N, D_HID), jnp.float32),
        pltpu.SemaphoreType.DMA,
        pltpu.SemaphoreType.DMA,
        pltpu.SemaphoreType.DMA,
        pltpu.SemaphoreType.DMA,
        pltpu.SemaphoreType.DMA,
        pltpu.SemaphoreType.DMA,
        pltpu.SemaphoreType.DMA,
        pltpu.SemaphoreType.DMA,
        pltpu.SemaphoreType.DMA,
        pltpu.SemaphoreType.DMA,
        pltpu.SemaphoreType.DMA,
        pltpu.SemaphoreType.DMA,
        pltpu.SemaphoreType.DMA,
    ],
)
def _prop_kernel(u_hbm, src_hbm, dst_hbm, zeros_hbm, out_hbm,
                 sidx_v, didx_v, rows_v, acc_sh, psem,
                 isem0, isem1, isem2, isem3,
                 gsem0, gsem1, gsem2, gsem3,
                 ssem0, ssem1, ssem2, ssem3):
    c = lax.axis_index("c")
    s = lax.axis_index("s")
    base = (c * NS + s) * EPT
    isem = (isem0, isem1, isem2, isem3)
    gsem = (gsem0, gsem1, gsem2, gsem3)
    ssem = (ssem0, ssem1, ssem2, ssem3)

    def idx_load(j, b):
        pltpu.async_copy(src_hbm.at[pl.ds(base + j * CH, CH)], sidx_v.at[b], isem[b])
        pltpu.async_copy(dst_hbm.at[pl.ds(base + j * CH, CH)], didx_v.at[b], isem[b])

    def wait_idx(j, b):
        pltpu.make_async_copy(src_hbm.at[pl.ds(base + j * CH, CH)], sidx_v.at[b], isem[b]).wait()
        pltpu.make_async_copy(dst_hbm.at[pl.ds(base + j * CH, CH)], didx_v.at[b], isem[b]).wait()

    def start_gather(b):
        pltpu.async_copy(u_hbm.at[sidx_v.at[b]], rows_v.at[b], gsem[b])

    def wait_gather(b):
        pltpu.make_async_copy(u_hbm.at[sidx_v.at[b]], rows_v.at[b], gsem[b]).wait()

    def start_scatter(b):
        pltpu.async_copy(rows_v.at[b], acc_sh.at[didx_v.at[b]], ssem[b], add=True)

    def wait_scatter(b):
        pltpu.make_async_copy(rows_v.at[b], acc_sh.at[didx_v.at[b]], ssem[b]).wait()

    def step(j, b, wait_sc, do_idx, do_g):
        if wait_sc:
            wait_scatter((b + 2) % NB)
        if do_idx:
            idx_load(j + 2, (b + 2) % NB)
        if do_g:
            wait_idx(j + 1, (b + 1) % NB)
            start_gather((b + 1) % NB)
        wait_gather(b)
        start_scatter(b)

    _rowwise_copy(s, lambda r0, nr: pltpu.async_copy(
        zeros_hbm.at[pl.ds(r0, nr)], acc_sh.at[pl.ds(r0, nr)], psem))
    idx_load(0, 0)
    idx_load(1, 1)
    wait_idx(0, 0)
    start_gather(0)
    _rowwise_copy(s, lambda r0, nr: pltpu.make_async_copy(
        zeros_hbm.at[pl.ds(r0, nr)], acc_sh.at[pl.ds(r0, nr)], psem).wait())
    plsc.subcore_barrier()

    step(0, 0, False, True, True)
    step(1, 1, False, True, True)
    step(2, 2, True, True, True)
    step(3, 3, True, True, True)

    def group(g, carry):
        j0 = g * NB
        for b in range(NB):
            step(j0 + b, b, True, True, True)
        return carry

    lax.fori_loop(1, NCH // NB - 1, group, 0)

    j0 = (NCH // NB - 1) * NB
    step(j0 + 0, 0, True, True, True)
    step(j0 + 1, 1, True, True, True)
    step(j0 + 2, 2, True, True, True)
    step(j0 + 3, 3, True, False, True)
    step(NCH - 1, (NCH - 1) % NB, True, False, False)
    wait_scatter((NCH - 2) % NB)
    wait_scatter((NCH - 1) % NB)
    plsc.subcore_barrier()
    _rowwise_copy(s, lambda r0, nr: pltpu.sync_copy(
        acc_sh.at[pl.ds(r0, nr)], out_hbm.at[c, pl.ds(r0, nr)]))


_R = 2000


def _dinv_block(d0_ref, d1_ref):
    return lax.rsqrt(d0_ref[...] + d1_ref[...] + 1.0)


def _mm1_body(x_ref, w_ref, d0_ref, d1_ref, u1_ref):
    dinv = _dinv_block(d0_ref, d1_ref)
    u1_ref[...] = jnp.dot(x_ref[...], w_ref[...],
                          preferred_element_type=jnp.float32) * dinv


def _mm2_body(p_ref, u1_ref, d0_ref, d1_ref, w_ref, b_ref, u2_ref):
    p = p_ref[...]
    dinv = _dinv_block(d0_ref, d1_ref)
    h = jnp.maximum((p[0] + p[1] + u1_ref[...]) * dinv + b_ref[...], 0.0)
    u2_ref[...] = jnp.dot(h, w_ref[...], preferred_element_type=jnp.float32) * dinv


def _out_body(p_ref, u2_ref, d0_ref, d1_ref, bmu_ref, bls_ref, mu_ref, ls_ref):
    p = p_ref[...]
    dinv = _dinv_block(d0_ref, d1_ref)
    o = (p[0] + p[1] + u2_ref[...]) * dinv
    mu_ref[...] = o[:, :D_OUT] + bmu_ref[...]
    ls_ref[...] = o[:, D_OUT:] + bls_ref[...]


def _row_spec(w):
    return pl.BlockSpec((_R, w), lambda i: (i, 0))


def _vec_spec():
    return pl.BlockSpec((_R, 1), lambda i: (i, 0))


def _full_spec(shape):
    nd = len(shape)
    return pl.BlockSpec(shape, lambda i: (0,) * nd)


def _part_spec(w):
    return pl.BlockSpec((NC, _R, w), lambda i: (0, i, 0))


_mm1 = pl.pallas_call(
    _mm1_body,
    grid=(N // _R,),
    in_specs=[_row_spec(D_IN), _full_spec((D_IN, D_HID)),
              _vec_spec(), _vec_spec()],
    out_specs=_row_spec(D_HID),
    out_shape=jax.ShapeDtypeStruct((N, D_HID), jnp.float32),
)

_mm2 = pl.pallas_call(
    _mm2_body,
    grid=(N // _R,),
    in_specs=[_part_spec(D_HID), _row_spec(D_HID), _vec_spec(), _vec_spec(),
              _full_spec((D_HID, 2 * D_OUT)), _full_spec((1, D_HID))],
    out_specs=_row_spec(2 * D_OUT),
    out_shape=jax.ShapeDtypeStruct((N, 2 * D_OUT), jnp.float32),
)

_out = pl.pallas_call(
    _out_body,
    grid=(N // _R,),
    in_specs=[_part_spec(2 * D_OUT), _row_spec(2 * D_OUT), _vec_spec(), _vec_spec(),
              _full_spec((1, D_OUT)), _full_spec((1, D_OUT))],
    out_specs=[_row_spec(D_OUT), _row_spec(D_OUT)],
    out_shape=[jax.ShapeDtypeStruct((N, D_OUT), jnp.float32),
               jax.ShapeDtypeStruct((N, D_OUT), jnp.float32)],
)


def kernel(x, edge_index, W1, b1, Wmu, bmu, Wls, bls):
    src = edge_index[0]
    dst = edge_index[1]
    zeros_acc = jnp.zeros((N, D_HID), jnp.float32)

    d0, d1 = _deg_kernel(dst)
    d0 = d0.reshape(N, 1)
    d1 = d1.reshape(N, 1)
    u1 = _mm1(x, W1, d0, d1)
    p1 = _prop_kernel(u1, src, dst, zeros_acc)
    wcat = jnp.concatenate([Wmu, Wls], axis=1)
    u2 = _mm2(p1, u1, d0, d1, wcat, b1.reshape(1, D_HID))
    p2 = _prop_kernel(u2, src, dst, zeros_acc)
    mu, logstd = _out(p2, u2, d0, d1, bmu.reshape(1, D_OUT), bls.reshape(1, D_OUT))
    return (mu, logstd)


# speedup vs baseline: 1.0435x; 1.0435x over previous
import functools

import jax
import jax.numpy as jnp
from jax import lax
from jax.experimental import pallas as pl
from jax.experimental.pallas import tpu as pltpu
from jax.experimental.pallas import tpu_sc as plsc

N = 10000
E = 320000
D_IN = 128
D_OUT = 64
D_HID = 128

NC = 2
NS = 16
NW = NC * NS
EPT = E // NW
CH = 80
NCH = EPT // CH
NB = 4
RPT = 624
RTAIL = N - NS * RPT


def _rowwise_copy(s, make_copy):
    make_copy(s * RPT, RPT)

    @pl.when(s == NS - 1)
    def _():
        make_copy(NS * RPT, RTAIL)


_MESH = plsc.VectorSubcoreMesh(core_axis_name="c", subcore_axis_name="s")


@functools.partial(
    pl.kernel,
    out_type=[jax.ShapeDtypeStruct((N,), jnp.float32),
              jax.ShapeDtypeStruct((N,), jnp.float32)],
    mesh=_MESH,
    scratch_types=[
        pltpu.VMEM((CH,), jnp.float32),
        pltpu.VMEM((NB, CH), jnp.int32),
        pltpu.VMEM((640,), jnp.float32),
        pltpu.VMEM_SHARED((N,), jnp.float32),
        pltpu.SemaphoreType.DMA,
        pltpu.SemaphoreType.DMA,
        pltpu.SemaphoreType.DMA,
        pltpu.SemaphoreType.DMA,
        pltpu.SemaphoreType.DMA,
        pltpu.SemaphoreType.DMA,
        pltpu.SemaphoreType.DMA,
        pltpu.SemaphoreType.DMA,
    ],
)
def _deg_kernel(dst_hbm, out0, out1, ones_v, didx_v, slab_v, acc_sh,
                isem0, isem1, isem2, isem3, ssem0, ssem1, ssem2, ssem3):
    c = lax.axis_index("c")
    s = lax.axis_index("s")
    base = (c * NS + s) * EPT
    isem = (isem0, isem1, isem2, isem3)
    ssem = (ssem0, ssem1, ssem2, ssem3)
    outs = (out0, out1)
    for k in range(CH // 16):
        ones_v[pl.ds(k * 16, 16)] = jnp.ones((16,), jnp.float32)
    for k in range(640 // 16):
        slab_v[pl.ds(k * 16, 16)] = jnp.zeros((16,), jnp.float32)

    def idx_load(j, b):
        pltpu.async_copy(dst_hbm.at[pl.ds(base + j * CH, CH)], didx_v.at[b], isem[b])

    def wait_idx(j, b):
        pltpu.make_async_copy(dst_hbm.at[pl.ds(base + j * CH, CH)],
                              didx_v.at[b], isem[b]).wait()

    def start_scatter(b):
        pltpu.async_copy(ones_v, acc_sh.at[didx_v.at[b]], ssem[b], add=True)

    def wait_scatter(b):
        pltpu.make_async_copy(ones_v, acc_sh.at[didx_v.at[b]], ssem[b]).wait()

    idx_load(0, 0)
    idx_load(1, 1)
    pltpu.sync_copy(slab_v.at[pl.ds(0, RPT)], acc_sh.at[pl.ds(s * RPT, RPT)])

    @pl.when(s == NS - 1)
    def _():
        pltpu.sync_copy(slab_v.at[pl.ds(0, RTAIL)], acc_sh.at[pl.ds(NS * RPT, RTAIL)])

    plsc.subcore_barrier()

    def chunk(j, carry):
        for b in range(NB):

            @pl.when(lax.rem(j, NB) == b)
            def _():
                bi = (b + 2) % NB

                @pl.when(j >= 2)
                def _():
                    wait_scatter(bi)

                @pl.when(j + 2 < NCH)
                def _():
                    idx_load(j + 2, bi)

                wait_idx(j, b)
                start_scatter(b)

        return carry

    lax.fori_loop(0, NCH, chunk, 0)
    wait_scatter((NCH - 1) % NB)
    wait_scatter((NCH - 2) % NB)
    plsc.subcore_barrier()
    for cc in range(NC):

        @pl.when(c == cc)
        def _():
            pltpu.sync_copy(acc_sh.at[pl.ds(s * RPT, RPT)],
                            slab_v.at[pl.ds(0, RPT)])
            pltpu.sync_copy(slab_v.at[pl.ds(0, RPT)],
                            outs[cc].at[pl.ds(s * RPT, RPT)])

            @pl.when(s == NS - 1)
            def _():
                pltpu.sync_copy(acc_sh.at[pl.ds(NS * RPT, RTAIL)],
                                slab_v.at[pl.ds(0, RTAIL)])
                pltpu.sync_copy(slab_v.at[pl.ds(0, RTAIL)],
                                outs[cc].at[pl.ds(NS * RPT, RTAIL)])


@functools.partial(
    pl.kernel,
    out_type=jax.ShapeDtypeStruct((NC, N, D_HID), jnp.float32),
    mesh=_MESH,
    scratch_types=[
        pltpu.VMEM((NB, CH), jnp.int32),
        pltpu.VMEM((NB, CH), jnp.int32),
        pltpu.VMEM((NB, CH, D_HID), jnp.float32),
        pltpu.VMEM_SHARED((N, D_HID), jnp.float32),
        pltpu.SemaphoreType.DMA,
        pltpu.SemaphoreType.DMA,
        pltpu.SemaphoreType.DMA,
        pltpu.SemaphoreType.DMA,
        pltpu.SemaphoreType.DMA,
        pltpu.SemaphoreType.DMA,
        pltpu.SemaphoreType.DMA,
        pltpu.SemaphoreType.DMA,
        pltpu.SemaphoreType.DMA,
        pltpu.SemaphoreType.DMA,
        pltpu.SemaphoreType.DMA,
        pltpu.SemaphoreType.DMA,
        pltpu.SemaphoreType.DMA,
    ],
)
def _prop_kernel(u_hbm, src_hbm, dst_hbm, zeros_hbm, out_hbm,
                 sidx_v, didx_v, rows_v, acc_sh, psem,
                 isem0, isem1, isem2, isem3,
                 gsem0, gsem1, gsem2, gsem3,
                 ssem0, ssem1, ssem2, ssem3):
    c = lax.axis_index("c")
    s = lax.axis_index("s")
    base = (c * NS + s) * EPT
    isem = (isem0, isem1, isem2, isem3)
    gsem = (gsem0, gsem1, gsem2, gsem3)
    ssem = (ssem0, ssem1, ssem2, ssem3)

    def idx_load(j, b):
        pltpu.async_copy(src_hbm.at[pl.ds(base + j * CH, CH)], sidx_v.at[b], isem[b])
        pltpu.async_copy(dst_hbm.at[pl.ds(base + j * CH, CH)], didx_v.at[b], isem[b])

    def wait_idx(j, b):
        pltpu.make_async_copy(src_hbm.at[pl.ds(base + j * CH, CH)], sidx_v.at[b], isem[b]).wait()
        pltpu.make_async_copy(dst_hbm.at[pl.ds(base + j * CH, CH)], didx_v.at[b], isem[b]).wait()

    def start_gather(b):
        pltpu.async_copy(u_hbm.at[sidx_v.at[b]], rows_v.at[b], gsem[b])

    def wait_gather(b):
        pltpu.make_async_copy(u_hbm.at[sidx_v.at[b]], rows_v.at[b], gsem[b]).wait()

    def start_scatter(b):
        pltpu.async_copy(rows_v.at[b], acc_sh.at[didx_v.at[b]], ssem[b], add=True)

    def wait_scatter(b):
        pltpu.make_async_copy(rows_v.at[b], acc_sh.at[didx_v.at[b]], ssem[b]).wait()

    def step(j, b, wait_sc, do_idx, do_g):
        if wait_sc:
            wait_scatter((b + 2) % NB)
        if do_idx:
            idx_load(j + 2, (b + 2) % NB)
        if do_g:
            wait_idx(j + 1, (b + 1) % NB)
            start_gather((b + 1) % NB)
        wait_gather(b)
        start_scatter(b)

    _rowwise_copy(s, lambda r0, nr: pltpu.async_copy(
        zeros_hbm.at[pl.ds(r0, nr)], acc_sh.at[pl.ds(r0, nr)], psem))
    idx_load(0, 0)
    idx_load(1, 1)
    wait_idx(0, 0)
    start_gather(0)
    _rowwise_copy(s, lambda r0, nr: pltpu.make_async_copy(
        zeros_hbm.at[pl.ds(r0, nr)], acc_sh.at[pl.ds(r0, nr)], psem).wait())
    plsc.subcore_barrier()

    step(0, 0, False, True, True)
    step(1, 1, False, True, True)
    step(2, 2, True, True, True)
    step(3, 3, True, True, True)

    def group(g, carry):
        j0 = g * NB
        for b in range(NB):
            step(j0 + b, b, True, True, True)
        return carry

    lax.fori_loop(1, NCH // NB - 1, group, 0)

    j0 = (NCH // NB - 1) * NB
    step(j0 + 0, 0, True, True, True)
    step(j0 + 1, 1, True, True, True)
    step(j0 + 2, 2, True, True, True)
    step(j0 + 3, 3, True, False, True)
    step(NCH - 1, (NCH - 1) % NB, True, False, False)
    wait_scatter((NCH - 2) % NB)
    wait_scatter((NCH - 1) % NB)
    plsc.subcore_barrier()
    _rowwise_copy(s, lambda r0, nr: pltpu.sync_copy(
        acc_sh.at[pl.ds(r0, nr)], out_hbm.at[c, pl.ds(r0, nr)]))


_R = 2000


def _dinv_block(d0_ref, d1_ref):
    return lax.rsqrt(d0_ref[...] + d1_ref[...] + 1.0)


def _mm1_body(x_ref, w_ref, d0_ref, d1_ref, u1_ref):
    dinv = _dinv_block(d0_ref, d1_ref)
    u1_ref[...] = jnp.dot(x_ref[...], w_ref[...],
                          preferred_element_type=jnp.float32) * dinv


def _mm2_body(p_ref, u1_ref, d0_ref, d1_ref, w_ref, b_ref, u2_ref):
    p = p_ref[...]
    dinv = _dinv_block(d0_ref, d1_ref)
    h = jnp.maximum((p[0] + p[1] + u1_ref[...]) * dinv + b_ref[...], 0.0)
    u2_ref[...] = jnp.dot(h, w_ref[...], preferred_element_type=jnp.float32) * dinv


def _out_body(p_ref, u2_ref, d0_ref, d1_ref, bmu_ref, bls_ref, mu_ref, ls_ref):
    p = p_ref[...]
    dinv = _dinv_block(d0_ref, d1_ref)
    o = (p[0] + p[1] + u2_ref[...]) * dinv
    mu_ref[...] = o[:, :D_OUT] + bmu_ref[...]
    ls_ref[...] = o[:, D_OUT:] + bls_ref[...]


def _row_spec(w):
    return pl.BlockSpec((_R, w), lambda i: (i, 0))


def _vec_spec():
    return pl.BlockSpec((_R, 1), lambda i: (i, 0))


def _full_spec(shape):
    nd = len(shape)
    return pl.BlockSpec(shape, lambda i: (0,) * nd)


def _part_spec(w):
    return pl.BlockSpec((NC, _R, w), lambda i: (0, i, 0))


_mm1 = pl.pallas_call(
    _mm1_body,
    grid=(N // _R,),
    in_specs=[_row_spec(D_IN), _full_spec((D_IN, D_HID)),
              _vec_spec(), _vec_spec()],
    out_specs=_row_spec(D_HID),
    out_shape=jax.ShapeDtypeStruct((N, D_HID), jnp.float32),
)

_mm2 = pl.pallas_call(
    _mm2_body,
    grid=(N // _R,),
    in_specs=[_part_spec(D_HID), _row_spec(D_HID), _vec_spec(), _vec_spec(),
              _full_spec((D_HID, 2 * D_OUT)), _full_spec((1, D_HID))],
    out_specs=_row_spec(2 * D_OUT),
    out_shape=jax.ShapeDtypeStruct((N, 2 * D_OUT), jnp.float32),
)

_out = pl.pallas_call(
    _out_body,
    grid=(N // _R,),
    in_specs=[_part_spec(2 * D_OUT), _row_spec(2 * D_OUT), _vec_spec(), _vec_spec(),
              _full_spec((1, D_OUT)), _full_spec((1, D_OUT))],
    out_specs=[_row_spec(D_OUT), _row_spec(D_OUT)],
    out_shape=[jax.ShapeDtypeStruct((N, D_OUT), jnp.float32),
               jax.ShapeDtypeStruct((N, D_OUT), jnp.float32)],
)


def kernel(x, edge_index, W1, b1, Wmu, bmu, Wls, bls):
    src = edge_index[0]
    dst = edge_index[1]
    zeros_acc = jnp.zeros((N, D_HID), jnp.float32)

    d0, d1 = _deg_kernel(dst)
    d0 = d0.reshape(N, 1)
    d1 = d1.reshape(N, 1)
    u1 = _mm1(x, W1, d0, d1)
    p1 = _prop_kernel(u1, src, dst, zeros_acc)
    wcat = jnp.concatenate([Wmu, Wls], axis=1)
    u2 = _mm2(p1, u1, d0, d1, wcat, b1.reshape(1, D_HID))
    p2 = _prop_kernel(u2, src, dst, zeros_acc)
    mu, logstd = _out(p2, u2, d0, d1, bmu.reshape(1, D_OUT), bls.reshape(1, D_OUT))
    return (mu, logstd)
